# Initial kernel scaffold; baseline (speedup 1.0000x reference)
import jax
import jax.numpy as jnp
from jax.experimental import pallas as pl


def kernel(atom_types, bond_types, edge_index, candidates, atom_embed, atom_W, atom_b, bond_embed, bond_W, bond_b, conv_W0, conv_b0, conv_W1, conv_b1, conv_W2, conv_b2, sc_W1, sc_b1, sc_W2, sc_b2):
    raise NotImplementedError("write your pallas kernel here")


# speedup vs baseline: 4.6507x; 4.6507x over previous
import functools

import jax
import jax.numpy as jnp
from jax import lax
from jax.experimental import pallas as pl
from jax.experimental.pallas import tpu as pltpu
from jax.experimental.pallas import tpu_sc as plsc

N = 10000
E = 320000
D = 128
P = 20000
PP = 20480
K = 128
NP = 10240
EC = E // K
NC = 2
NS = 16
NW = NC * NS
RPT = NP // NS
BR = 1000
BP = 1024

_mesh = plsc.VectorSubcoreMesh(core_axis_name="c", subcore_axis_name="s")


@functools.partial(
    pl.kernel,
    mesh=_mesh,
    out_type=jax.ShapeDtypeStruct((NC, NP, D), jnp.float32),
    scratch_types=[
        pltpu.VMEM((K,), jnp.int32),
        pltpu.VMEM((K,), jnp.int32),
        pltpu.VMEM((K, D), jnp.float32),
        pltpu.VMEM((K, D), jnp.float32),
        pltpu.VMEM_SHARED((NP, D), jnp.float32),
    ],
)
def _sc_degrees(src_hbm, dst_hbm, ones_s_hbm, ones_d_hbm, zeros_hbm, out_hbm,
                idx_s, idx_d, ones_s, ones_d, deg):
    c = lax.axis_index("c")
    s = lax.axis_index("s")
    wid = s * NC + c
    r0 = s * RPT
    pltpu.sync_copy(zeros_hbm.at[pl.ds(r0, RPT)], deg.at[pl.ds(r0, RPT)])
    pltpu.sync_copy(ones_s_hbm, ones_s)
    pltpu.sync_copy(ones_d_hbm, ones_d)
    plsc.subcore_barrier()

    def body(t, carry):
        ch = wid + t * NW

        @pl.when(ch < EC)
        def _():
            e0 = ch * K
            pltpu.sync_copy(src_hbm.at[pl.ds(e0, K)], idx_s)
            pltpu.sync_copy(dst_hbm.at[pl.ds(e0, K)], idx_d)
            pltpu.sync_copy(ones_s, deg.at[idx_s], add=True)
            pltpu.sync_copy(ones_d, deg.at[idx_d], add=True)

        return carry

    lax.fori_loop(0, (EC + NW - 1) // NW, body, 0)
    plsc.subcore_barrier()
    pltpu.sync_copy(deg.at[pl.ds(r0, RPT)], out_hbm.at[c, pl.ds(r0, RPT)])


@functools.partial(
    pl.kernel,
    mesh=_mesh,
    out_type=jax.ShapeDtypeStruct((NC, NP, D), jnp.float32),
    scratch_types=[
        pltpu.VMEM((K,), jnp.int32),
        pltpu.VMEM((K,), jnp.int32),
        pltpu.VMEM((K, D), jnp.float32),
        pltpu.SemaphoreType.DMA,
        pltpu.VMEM_SHARED((NP, D), jnp.float32),
    ],
)
def _sc_aggregate(hs_hbm, src_hbm, dst_hbm, zeros_hbm, out_hbm,
                  idx_s, idx_d, rows, sem, acc):
    c = lax.axis_index("c")
    s = lax.axis_index("s")
    wid = s * NC + c
    r0 = s * RPT
    pltpu.sync_copy(zeros_hbm.at[pl.ds(r0, RPT)], acc.at[pl.ds(r0, RPT)])
    plsc.subcore_barrier()

    def body(t, carry):
        ch = wid + t * NW

        @pl.when(ch < EC)
        def _():
            e0 = ch * K
            pltpu.sync_copy(src_hbm.at[pl.ds(e0, K)], idx_s)
            pltpu.sync_copy(dst_hbm.at[pl.ds(e0, K)], idx_d)
            pltpu.async_copy(hs_hbm.at[idx_s], rows, sem).wait()
            pltpu.sync_copy(rows, acc.at[idx_d], add=True)

        return carry

    lax.fori_loop(0, (EC + NW - 1) // NW, body, 0)
    plsc.subcore_barrier()
    pltpu.sync_copy(acc.at[pl.ds(r0, RPT)], out_hbm.at[c, pl.ds(r0, RPT)])


@functools.partial(
    pl.kernel,
    mesh=_mesh,
    out_type=(jax.ShapeDtypeStruct((PP, D), jnp.float32),
              jax.ShapeDtypeStruct((PP, D), jnp.float32)),
    scratch_types=[
        pltpu.VMEM((K,), jnp.int32),
        pltpu.VMEM((K, D), jnp.float32),
        pltpu.SemaphoreType.DMA,
    ],
)
def _sc_pair_gather(ha_hbm, hb_hbm, u_hbm, v_hbm, outa_hbm, outb_hbm,
                    idx, rows, sem):
    c = lax.axis_index("c")
    s = lax.axis_index("s")
    wid = s * NC + c

    def body(t, carry):
        p0 = (wid + t * NW) * K
        pltpu.sync_copy(u_hbm.at[pl.ds(p0, K)], idx)
        pltpu.async_copy(ha_hbm.at[idx], rows, sem).wait()
        pltpu.sync_copy(rows, outa_hbm.at[pl.ds(p0, K)])
        pltpu.sync_copy(v_hbm.at[pl.ds(p0, K)], idx)
        pltpu.async_copy(hb_hbm.at[idx], rows, sem).wait()
        pltpu.sync_copy(rows, outb_hbm.at[pl.ds(p0, K)])
        return carry

    lax.fori_loop(0, PP // K // NW, body, 0)


def _rsqrt_deg(a_ref, b_ref, col):
    deg = a_ref[:, col:col + 1] + b_ref[:, col:col + 1]
    return lax.rsqrt(jnp.maximum(deg, 1.0))


def _enc_body(t_ref, emb_ref, aw_ref, ab_ref, d0_ref, d1_ref, o_ref):
    t = t_ref[...]
    oh = (t == lax.broadcasted_iota(jnp.int32, (BR, D), 1)).astype(jnp.float32)
    embw = jnp.dot(emb_ref[...], aw_ref[...], preferred_element_type=jnp.float32)
    h = jnp.dot(oh, embw, preferred_element_type=jnp.float32) + ab_ref[...]
    o_ref[...] = h * _rsqrt_deg(d0_ref, d1_ref, 0)


def _layer_body(p0_ref, p1_ref, d0_ref, d1_ref, w_ref, b_ref, o_ref):
    agg = (p0_ref[...] + p1_ref[...]) * _rsqrt_deg(d0_ref, d1_ref, 64)
    h = jnp.dot(agg, w_ref[...], preferred_element_type=jnp.float32) + b_ref[...]
    h = jnp.maximum(h, 0.0)
    o_ref[...] = h * _rsqrt_deg(d0_ref, d1_ref, 0)


def _layer3_body(p0_ref, p1_ref, d0_ref, d1_ref, w_ref, b_ref,
                 w1a_ref, w1b_ref, b1_ref, oa_ref, ob_ref):
    agg = (p0_ref[...] + p1_ref[...]) * _rsqrt_deg(d0_ref, d1_ref, 64)
    h = jnp.dot(agg, w_ref[...], preferred_element_type=jnp.float32) + b_ref[...]
    h = jnp.maximum(h, 0.0)
    oa_ref[...] = jnp.dot(h, w1a_ref[...], preferred_element_type=jnp.float32) + b1_ref[...]
    ob_ref[...] = jnp.dot(h, w1b_ref[...], preferred_element_type=jnp.float32)


def _score_body(a_ref, b_ref, w2_ref, b2_ref, o_ref):
    hid = jnp.maximum(a_ref[...] + b_ref[...], 0.0)
    s = jnp.sum(hid * w2_ref[...], axis=1, keepdims=True)
    o_ref[...] = s + b2_ref[:, 0:1]


def _row_spec(bs):
    return pl.BlockSpec(bs, lambda i: (i, 0))


def _full_spec(shape):
    return pl.BlockSpec(shape, lambda i: (0, 0))


def _tc_encoder(types2, emb_pad, atom_W, atom_b2, dg0, dg1):
    return pl.pallas_call(
        _enc_body,
        grid=(N // BR,),
        in_specs=[_row_spec((BR, 1)), _full_spec((D, D)), _full_spec((D, D)),
                  _full_spec((1, D)), _row_spec((BR, D)), _row_spec((BR, D))],
        out_specs=_row_spec((BR, D)),
        out_shape=jax.ShapeDtypeStruct((N, D), jnp.float32),
    )(types2, emb_pad, atom_W, atom_b2, dg0, dg1)


def _tc_layer(p0, p1, dg0, dg1, w, b2):
    return pl.pallas_call(
        _layer_body,
        grid=(N // BR,),
        in_specs=[_row_spec((BR, D)), _row_spec((BR, D)),
                  _row_spec((BR, D)), _row_spec((BR, D)),
                  _full_spec((D, D)), _full_spec((1, D))],
        out_specs=_row_spec((BR, D)),
        out_shape=jax.ShapeDtypeStruct((N, D), jnp.float32),
    )(p0, p1, dg0, dg1, w, b2)


def _tc_layer3(p0, p1, dg0, dg1, w, b2, w1a, w1b, b12):
    return pl.pallas_call(
        _layer3_body,
        grid=(N // BR,),
        in_specs=[_row_spec((BR, D)), _row_spec((BR, D)),
                  _row_spec((BR, D)), _row_spec((BR, D)),
                  _full_spec((D, D)), _full_spec((1, D)),
                  _full_spec((D, D)), _full_spec((D, D)), _full_spec((1, D))],
        out_specs=(_row_spec((BR, D)), _row_spec((BR, D))),
        out_shape=(jax.ShapeDtypeStruct((N, D), jnp.float32),
                   jax.ShapeDtypeStruct((N, D), jnp.float32)),
    )(p0, p1, dg0, dg1, w, b2, w1a, w1b, b12)


def _tc_score(hag, hbg, w2row, b2row):
    return pl.pallas_call(
        _score_body,
        grid=(PP // BP,),
        in_specs=[_row_spec((BP, D)), _row_spec((BP, D)),
                  _full_spec((1, D)), _full_spec((1, D))],
        out_specs=_row_spec((BP, 1)),
        out_shape=jax.ShapeDtypeStruct((PP, 1), jnp.float32),
    )(hag, hbg, w2row, b2row)


def kernel(atom_types, bond_types, edge_index, candidates,
           atom_embed, atom_W, atom_b,
           bond_embed, bond_W, bond_b,
           conv_W0, conv_b0, conv_W1, conv_b1, conv_W2, conv_b2,
           sc_W1, sc_b1, sc_W2, sc_b2):
    f32 = jnp.float32
    src = edge_index[0].astype(jnp.int32)
    dst = edge_index[1].astype(jnp.int32)
    types2 = atom_types.astype(jnp.int32).reshape(N, 1)
    emb_pad = jnp.pad(atom_embed, ((0, D - atom_embed.shape[0]), (0, 0)))

    zerosND = jnp.zeros((NP, D), f32)
    col = jnp.arange(D, dtype=jnp.int32)
    ones_s = jnp.broadcast_to((col == 0).astype(f32), (K, D))
    ones_d = jnp.broadcast_to((col == 64).astype(f32), (K, D))

    degp = _sc_degrees(src, dst, ones_s, ones_d, zerosND)
    dg0, dg1 = degp[0], degp[1]

    h = _tc_encoder(types2, emb_pad, atom_W, atom_b.reshape(1, D), dg0, dg1)

    agg = _sc_aggregate(h, src, dst, zerosND)
    h = _tc_layer(agg[0], agg[1], dg0, dg1, conv_W0, conv_b0.reshape(1, D))
    agg = _sc_aggregate(h, src, dst, zerosND)
    h = _tc_layer(agg[0], agg[1], dg0, dg1, conv_W1, conv_b1.reshape(1, D))
    agg = _sc_aggregate(h, src, dst, zerosND)

    ha, hb = _tc_layer3(agg[0], agg[1], dg0, dg1,
                        conv_W2, conv_b2.reshape(1, D),
                        sc_W1[:D], sc_W1[D:], sc_b1.reshape(1, D))

    pad = (jnp.arange(PP - P, dtype=jnp.int32) * 37) % N
    u = jnp.concatenate([candidates[:, 0].astype(jnp.int32), pad])
    v = jnp.concatenate([candidates[:, 1].astype(jnp.int32), pad])

    hag, hbg = _sc_pair_gather(ha, hb, u, v)

    w2row = sc_W2.reshape(1, D)
    b2row = jnp.broadcast_to(sc_b2.reshape(1, 1), (1, D))
    scores = _tc_score(hag, hbg, w2row, b2row)
    return scores[:P, 0]


# speedup vs baseline: 7.4559x; 1.6032x over previous
import functools

import jax
import jax.numpy as jnp
from jax import lax
from jax.experimental import pallas as pl
from jax.experimental.pallas import tpu as pltpu
from jax.experimental.pallas import tpu_sc as plsc

N = 10000
E = 320000
D = 128
P = 20000
PP = 20480
K = 128
NP = 10112
EC = E // K
TW = 78
NC = 2
NS = 16
NW = NC * NS
RPT = NP // NS
BR = 1000
BP = 1024

_mesh = plsc.VectorSubcoreMesh(core_axis_name="c", subcore_axis_name="s")


@functools.partial(
    pl.kernel,
    mesh=_mesh,
    out_type=jax.ShapeDtypeStruct((NC, NP, D), jnp.float32),
    scratch_types=(
        [pltpu.VMEM((K,), jnp.int32)] * 12
        + [pltpu.VMEM((K, D), jnp.float32)] * 2
        + [pltpu.SemaphoreType.DMA] * 12
        + [pltpu.VMEM_SHARED((NP, D), jnp.float32)]
    ),
)
def _sc_degrees(src_hbm, dst_hbm, ones_s_hbm, ones_d_hbm, zeros_hbm, out_hbm,
                is0, is1, is2, is3, is4, is5, id0, id1, id2, id3, id4, id5,
                ones_s, ones_d,
                im0, im1, im2, im3, im4, im5, sm0, sm1, sm2, dm0, dm1, dm2,
                deg):
    c = lax.axis_index("c")
    s = lax.axis_index("s")
    wid = s * NC + c
    r0 = s * RPT
    isx = (is0, is1, is2, is3, is4, is5)
    idx = (id0, id1, id2, id3, id4, id5)
    isem = (im0, im1, im2, im3, im4, im5)
    ssem = (sm0, sm1, sm2)
    dsem = (dm0, dm1, dm2)
    pltpu.sync_copy(zeros_hbm.at[pl.ds(r0, RPT)], deg.at[pl.ds(r0, RPT)])
    pltpu.sync_copy(ones_s_hbm, ones_s)
    pltpu.sync_copy(ones_d_hbm, ones_d)
    plsc.subcore_barrier()

    def e0_of(t):
        return (wid * TW + t) * K

    def load_idx(t, q):
        pltpu.async_copy(src_hbm.at[pl.ds(e0_of(t), K)], isx[q], isem[q])
        pltpu.async_copy(dst_hbm.at[pl.ds(e0_of(t), K)], idx[q], isem[q])

    def wait_idx(q):
        pltpu.make_async_copy(src_hbm.at[pl.ds(0, K)], isx[q], isem[q]).wait()
        pltpu.make_async_copy(dst_hbm.at[pl.ds(0, K)], idx[q], isem[q]).wait()

    def start_scatter(r, q):
        pltpu.async_copy(ones_s, deg.at[isx[q]], ssem[r], add=True)
        pltpu.async_copy(ones_d, deg.at[idx[q]], dsem[r], add=True)

    def wait_scatter(r, q):
        pltpu.make_async_copy(ones_s, deg.at[isx[q]], ssem[r]).wait()
        pltpu.make_async_copy(ones_d, deg.at[idx[q]], dsem[r]).wait()

    for q in range(4):
        load_idx(q, q)

    def body(i, carry):
        for k in range(6):
            t = 6 * i + k
            q = k
            r = k % 3
            wait_idx(q)
            start_scatter(r, q)

            @pl.when(t >= 2)
            def _():
                wait_scatter((k + 1) % 3, (k + 4) % 6)

            @pl.when(t + 4 < TW)
            def _():
                load_idx(t + 4, (k + 4) % 6)

        return carry

    lax.fori_loop(0, TW // 6, body, 0)
    wait_scatter((TW - 2) % 3, (TW - 2) % 6)
    wait_scatter((TW - 1) % 3, (TW - 1) % 6)

    @pl.when(wid < EC - TW * NW)
    def _():
        e0 = (TW * NW + wid) * K
        pltpu.sync_copy(src_hbm.at[pl.ds(e0, K)], is0)
        pltpu.sync_copy(dst_hbm.at[pl.ds(e0, K)], id0)
        pltpu.sync_copy(ones_s, deg.at[is0], add=True)
        pltpu.sync_copy(ones_d, deg.at[id0], add=True)

    plsc.subcore_barrier()
    pltpu.sync_copy(deg.at[pl.ds(r0, RPT)], out_hbm.at[c, pl.ds(r0, RPT)])


@functools.partial(
    pl.kernel,
    mesh=_mesh,
    out_type=jax.ShapeDtypeStruct((NC, NP, D), jnp.float32),
    scratch_types=(
        [pltpu.VMEM((K,), jnp.int32)] * 12
        + [pltpu.VMEM((K, D), jnp.float32)] * 2
        + [pltpu.SemaphoreType.DMA] * 10
        + [pltpu.VMEM_SHARED((NP, D), jnp.float32)]
    ),
)
def _sc_aggregate(hs_hbm, src_hbm, dst_hbm, zeros_hbm, out_hbm,
                  is0, is1, is2, is3, is4, is5, id0, id1, id2, id3, id4, id5,
                  rows0, rows1,
                  im0, im1, im2, im3, im4, im5, gm0, gm1, sm0, sm1,
                  acc):
    c = lax.axis_index("c")
    s = lax.axis_index("s")
    wid = s * NC + c
    r0 = s * RPT
    isx = (is0, is1, is2, is3, is4, is5)
    idx = (id0, id1, id2, id3, id4, id5)
    rows = (rows0, rows1)
    isem = (im0, im1, im2, im3, im4, im5)
    gsem = (gm0, gm1)
    ssem = (sm0, sm1)
    pltpu.sync_copy(zeros_hbm.at[pl.ds(r0, RPT)], acc.at[pl.ds(r0, RPT)])
    plsc.subcore_barrier()

    def e0_of(t):
        return (wid * TW + t) * K

    def load_idx(t, q):
        pltpu.async_copy(src_hbm.at[pl.ds(e0_of(t), K)], isx[q], isem[q])
        pltpu.async_copy(dst_hbm.at[pl.ds(e0_of(t), K)], idx[q], isem[q])

    def wait_idx(q):
        pltpu.make_async_copy(src_hbm.at[pl.ds(0, K)], isx[q], isem[q]).wait()
        pltpu.make_async_copy(dst_hbm.at[pl.ds(0, K)], idx[q], isem[q]).wait()

    def start_gather(r, q):
        pltpu.async_copy(hs_hbm.at[isx[q]], rows[r], gsem[r])

    def wait_gather(r, q):
        pltpu.make_async_copy(hs_hbm.at[isx[q]], rows[r], gsem[r]).wait()

    def start_scatter(r, q):
        pltpu.async_copy(rows[r], acc.at[idx[q]], ssem[r], add=True)

    def wait_scatter(r, q):
        pltpu.make_async_copy(rows[r], acc.at[idx[q]], ssem[r]).wait()

    for q in range(4):
        load_idx(q, q)
    wait_idx(0)
    start_gather(0, 0)

    def body(i, carry):
        for k in range(6):
            t = 6 * i + k
            q = k
            r = k % 2
            qn = (k + 1) % 6
            rn = (k + 1) % 2
            wait_gather(r, q)
            start_scatter(r, q)

            @pl.when(t >= 1)
            def _():
                wait_scatter(rn, (k + 5) % 6)

            @pl.when(t + 1 < TW)
            def _():
                wait_idx(qn)
                start_gather(rn, qn)

            @pl.when(t + 4 < TW)
            def _():
                load_idx(t + 4, (k + 4) % 6)

        return carry

    lax.fori_loop(0, TW // 6, body, 0)
    wait_scatter((TW - 1) % 2, (TW - 1) % 6)

    @pl.when(wid < EC - TW * NW)
    def _():
        e0 = (TW * NW + wid) * K
        pltpu.sync_copy(src_hbm.at[pl.ds(e0, K)], is0)
        pltpu.sync_copy(dst_hbm.at[pl.ds(e0, K)], id0)
        pltpu.async_copy(hs_hbm.at[is0], rows0, gm0).wait()
        pltpu.sync_copy(rows0, acc.at[id0], add=True)

    plsc.subcore_barrier()
    pltpu.sync_copy(acc.at[pl.ds(r0, RPT)], out_hbm.at[c, pl.ds(r0, RPT)])


@functools.partial(
    pl.kernel,
    mesh=_mesh,
    out_type=(jax.ShapeDtypeStruct((PP, D), jnp.float32),
              jax.ShapeDtypeStruct((PP, D), jnp.float32)),
    scratch_types=[
        pltpu.VMEM((K,), jnp.int32),
        pltpu.VMEM((K, D), jnp.float32),
        pltpu.SemaphoreType.DMA,
    ],
)
def _sc_pair_gather(ha_hbm, hb_hbm, u_hbm, v_hbm, outa_hbm, outb_hbm,
                    idx, rows, sem):
    c = lax.axis_index("c")
    s = lax.axis_index("s")
    wid = s * NC + c

    def body(t, carry):
        p0 = (wid + t * NW) * K
        pltpu.sync_copy(u_hbm.at[pl.ds(p0, K)], idx)
        pltpu.async_copy(ha_hbm.at[idx], rows, sem).wait()
        pltpu.sync_copy(rows, outa_hbm.at[pl.ds(p0, K)])
        pltpu.sync_copy(v_hbm.at[pl.ds(p0, K)], idx)
        pltpu.async_copy(hb_hbm.at[idx], rows, sem).wait()
        pltpu.sync_copy(rows, outb_hbm.at[pl.ds(p0, K)])
        return carry

    lax.fori_loop(0, PP // K // NW, body, 0)


def _rsqrt_deg(a_ref, b_ref, col):
    deg = a_ref[:, col:col + 1] + b_ref[:, col:col + 1]
    return lax.rsqrt(jnp.maximum(deg, 1.0))


def _enc_body(t_ref, emb_ref, aw_ref, ab_ref, d0_ref, d1_ref, o_ref):
    t = t_ref[...]
    oh = (t == lax.broadcasted_iota(jnp.int32, (BR, D), 1)).astype(jnp.float32)
    embw = jnp.dot(emb_ref[...], aw_ref[...], preferred_element_type=jnp.float32)
    h = jnp.dot(oh, embw, preferred_element_type=jnp.float32) + ab_ref[...]
    o_ref[...] = h * _rsqrt_deg(d0_ref, d1_ref, 0)


def _layer_body(p0_ref, p1_ref, d0_ref, d1_ref, w_ref, b_ref, o_ref):
    agg = (p0_ref[...] + p1_ref[...]) * _rsqrt_deg(d0_ref, d1_ref, 64)
    h = jnp.dot(agg, w_ref[...], preferred_element_type=jnp.float32) + b_ref[...]
    h = jnp.maximum(h, 0.0)
    o_ref[...] = h * _rsqrt_deg(d0_ref, d1_ref, 0)


def _layer3_body(p0_ref, p1_ref, d0_ref, d1_ref, w_ref, b_ref,
                 w1a_ref, w1b_ref, b1_ref, oa_ref, ob_ref):
    agg = (p0_ref[...] + p1_ref[...]) * _rsqrt_deg(d0_ref, d1_ref, 64)
    h = jnp.dot(agg, w_ref[...], preferred_element_type=jnp.float32) + b_ref[...]
    h = jnp.maximum(h, 0.0)
    oa_ref[...] = jnp.dot(h, w1a_ref[...], preferred_element_type=jnp.float32) + b1_ref[...]
    ob_ref[...] = jnp.dot(h, w1b_ref[...], preferred_element_type=jnp.float32)


def _score_body(a_ref, b_ref, w2_ref, b2_ref, o_ref):
    hid = jnp.maximum(a_ref[...] + b_ref[...], 0.0)
    s = jnp.sum(hid * w2_ref[...], axis=1, keepdims=True)
    o_ref[...] = s + b2_ref[:, 0:1]


def _row_spec(bs):
    return pl.BlockSpec(bs, lambda i: (i, 0))


def _full_spec(shape):
    return pl.BlockSpec(shape, lambda i: (0, 0))


def _tc_encoder(types2, emb_pad, atom_W, atom_b2, dg0, dg1):
    return pl.pallas_call(
        _enc_body,
        grid=(N // BR,),
        in_specs=[_row_spec((BR, 1)), _full_spec((D, D)), _full_spec((D, D)),
                  _full_spec((1, D)), _row_spec((BR, D)), _row_spec((BR, D))],
        out_specs=_row_spec((BR, D)),
        out_shape=jax.ShapeDtypeStruct((N, D), jnp.float32),
    )(types2, emb_pad, atom_W, atom_b2, dg0, dg1)


def _tc_layer(p0, p1, dg0, dg1, w, b2):
    return pl.pallas_call(
        _layer_body,
        grid=(N // BR,),
        in_specs=[_row_spec((BR, D)), _row_spec((BR, D)),
                  _row_spec((BR, D)), _row_spec((BR, D)),
                  _full_spec((D, D)), _full_spec((1, D))],
        out_specs=_row_spec((BR, D)),
        out_shape=jax.ShapeDtypeStruct((N, D), jnp.float32),
    )(p0, p1, dg0, dg1, w, b2)


def _tc_layer3(p0, p1, dg0, dg1, w, b2, w1a, w1b, b12):
    return pl.pallas_call(
        _layer3_body,
        grid=(N // BR,),
        in_specs=[_row_spec((BR, D)), _row_spec((BR, D)),
                  _row_spec((BR, D)), _row_spec((BR, D)),
                  _full_spec((D, D)), _full_spec((1, D)),
                  _full_spec((D, D)), _full_spec((D, D)), _full_spec((1, D))],
        out_specs=(_row_spec((BR, D)), _row_spec((BR, D))),
        out_shape=(jax.ShapeDtypeStruct((N, D), jnp.float32),
                   jax.ShapeDtypeStruct((N, D), jnp.float32)),
    )(p0, p1, dg0, dg1, w, b2, w1a, w1b, b12)


def _tc_score(hag, hbg, w2row, b2row):
    return pl.pallas_call(
        _score_body,
        grid=(PP // BP,),
        in_specs=[_row_spec((BP, D)), _row_spec((BP, D)),
                  _full_spec((1, D)), _full_spec((1, D))],
        out_specs=_row_spec((BP, 1)),
        out_shape=jax.ShapeDtypeStruct((PP, 1), jnp.float32),
    )(hag, hbg, w2row, b2row)


def kernel(atom_types, bond_types, edge_index, candidates,
           atom_embed, atom_W, atom_b,
           bond_embed, bond_W, bond_b,
           conv_W0, conv_b0, conv_W1, conv_b1, conv_W2, conv_b2,
           sc_W1, sc_b1, sc_W2, sc_b2):
    f32 = jnp.float32
    src = edge_index[0].astype(jnp.int32)
    dst = edge_index[1].astype(jnp.int32)
    types2 = atom_types.astype(jnp.int32).reshape(N, 1)
    emb_pad = jnp.pad(atom_embed, ((0, D - atom_embed.shape[0]), (0, 0)))

    zerosND = jnp.zeros((NP, D), f32)
    col = jnp.arange(D, dtype=jnp.int32)
    ones_s = jnp.broadcast_to((col == 0).astype(f32), (K, D))
    ones_d = jnp.broadcast_to((col == 64).astype(f32), (K, D))

    degp = _sc_degrees(src, dst, ones_s, ones_d, zerosND)
    dg0, dg1 = degp[0], degp[1]

    h = _tc_encoder(types2, emb_pad, atom_W, atom_b.reshape(1, D), dg0, dg1)

    agg = _sc_aggregate(h, src, dst, zerosND)
    h = _tc_layer(agg[0], agg[1], dg0, dg1, conv_W0, conv_b0.reshape(1, D))
    agg = _sc_aggregate(h, src, dst, zerosND)
    h = _tc_layer(agg[0], agg[1], dg0, dg1, conv_W1, conv_b1.reshape(1, D))
    agg = _sc_aggregate(h, src, dst, zerosND)

    ha, hb = _tc_layer3(agg[0], agg[1], dg0, dg1,
                        conv_W2, conv_b2.reshape(1, D),
                        sc_W1[:D], sc_W1[D:], sc_b1.reshape(1, D))

    pad = (jnp.arange(PP - P, dtype=jnp.int32) * 37) % N
    u = jnp.concatenate([candidates[:, 0].astype(jnp.int32), pad])
    v = jnp.concatenate([candidates[:, 1].astype(jnp.int32), pad])

    hag, hbg = _sc_pair_gather(ha, hb, u, v)

    w2row = sc_W2.reshape(1, D)
    b2row = jnp.broadcast_to(sc_b2.reshape(1, 1), (1, D))
    scores = _tc_score(hag, hbg, w2row, b2row)
    return scores[:P, 0]


# speedup vs baseline: 8.4167x; 1.1289x over previous
import functools

import jax
import jax.numpy as jnp
from jax import lax
from jax.experimental import pallas as pl
from jax.experimental.pallas import tpu as pltpu
from jax.experimental.pallas import tpu_sc as plsc

N = 10000
E = 320000
D = 128
P = 20000
PP = 20480
K = 128
NP = 10112
EC = E // K
TW = 78
NC = 2
NS = 16
NW = NC * NS
PCW = PP // K // NW
RPT = NP // NS
BR = 1000
BP = 1024

_mesh = plsc.VectorSubcoreMesh(core_axis_name="c", subcore_axis_name="s")


@functools.partial(
    pl.kernel,
    mesh=_mesh,
    out_type=jax.ShapeDtypeStruct((NC, NP, D), jnp.float32),
    scratch_types=(
        [pltpu.VMEM((K,), jnp.int32)] * 12
        + [pltpu.VMEM((K, D), jnp.float32)] * 2
        + [pltpu.SemaphoreType.DMA] * 12
        + [pltpu.VMEM_SHARED((NP, D), jnp.float32)]
    ),
)
def _sc_degrees(src_hbm, dst_hbm, ones_s_hbm, ones_d_hbm, zeros_hbm, out_hbm,
                is0, is1, is2, is3, is4, is5, id0, id1, id2, id3, id4, id5,
                ones_s, ones_d,
                im0, im1, im2, im3, im4, im5, sm0, sm1, sm2, dm0, dm1, dm2,
                deg):
    c = lax.axis_index("c")
    s = lax.axis_index("s")
    wid = s * NC + c
    r0 = s * RPT
    isx = (is0, is1, is2, is3, is4, is5)
    idx = (id0, id1, id2, id3, id4, id5)
    isem = (im0, im1, im2, im3, im4, im5)
    ssem = (sm0, sm1, sm2)
    dsem = (dm0, dm1, dm2)
    pltpu.sync_copy(zeros_hbm.at[pl.ds(r0, RPT)], deg.at[pl.ds(r0, RPT)])
    pltpu.sync_copy(ones_s_hbm, ones_s)
    pltpu.sync_copy(ones_d_hbm, ones_d)
    plsc.subcore_barrier()

    def e0_of(t):
        return (wid * TW + t) * K

    def load_idx(t, q):
        pltpu.async_copy(src_hbm.at[pl.ds(e0_of(t), K)], isx[q], isem[q])
        pltpu.async_copy(dst_hbm.at[pl.ds(e0_of(t), K)], idx[q], isem[q])

    def wait_idx(q):
        pltpu.make_async_copy(src_hbm.at[pl.ds(0, K)], isx[q], isem[q]).wait()
        pltpu.make_async_copy(dst_hbm.at[pl.ds(0, K)], idx[q], isem[q]).wait()

    def start_scatter(r, q):
        pltpu.async_copy(ones_s, deg.at[isx[q]], ssem[r], add=True)
        pltpu.async_copy(ones_d, deg.at[idx[q]], dsem[r], add=True)

    def wait_scatter(r, q):
        pltpu.make_async_copy(ones_s, deg.at[isx[q]], ssem[r]).wait()
        pltpu.make_async_copy(ones_d, deg.at[idx[q]], dsem[r]).wait()

    for q in range(4):
        load_idx(q, q)

    def body(i, carry):
        for k in range(6):
            t = 6 * i + k
            q = k
            r = k % 3
            wait_idx(q)
            start_scatter(r, q)

            @pl.when(t >= 2)
            def _():
                wait_scatter((k + 1) % 3, (k + 4) % 6)

            @pl.when(t + 4 < TW)
            def _():
                load_idx(t + 4, (k + 4) % 6)

        return carry

    lax.fori_loop(0, TW // 6, body, 0)
    wait_scatter((TW - 2) % 3, (TW - 2) % 6)
    wait_scatter((TW - 1) % 3, (TW - 1) % 6)

    @pl.when(wid < EC - TW * NW)
    def _():
        e0 = (TW * NW + wid) * K
        pltpu.sync_copy(src_hbm.at[pl.ds(e0, K)], is0)
        pltpu.sync_copy(dst_hbm.at[pl.ds(e0, K)], id0)
        pltpu.sync_copy(ones_s, deg.at[is0], add=True)
        pltpu.sync_copy(ones_d, deg.at[id0], add=True)

    plsc.subcore_barrier()
    pltpu.sync_copy(deg.at[pl.ds(r0, RPT)], out_hbm.at[c, pl.ds(r0, RPT)])


@functools.partial(
    pl.kernel,
    mesh=_mesh,
    out_type=jax.ShapeDtypeStruct((NC, NP, D), jnp.float32),
    scratch_types=(
        [pltpu.VMEM((K,), jnp.int32)] * 12
        + [pltpu.VMEM((K, D), jnp.float32)] * 2
        + [pltpu.SemaphoreType.DMA] * 10
        + [pltpu.VMEM_SHARED((NP, D), jnp.float32)]
    ),
)
def _sc_aggregate(hs_hbm, src_hbm, dst_hbm, zeros_hbm, out_hbm,
                  is0, is1, is2, is3, is4, is5, id0, id1, id2, id3, id4, id5,
                  rows0, rows1,
                  im0, im1, im2, im3, im4, im5, gm0, gm1, sm0, sm1,
                  acc):
    c = lax.axis_index("c")
    s = lax.axis_index("s")
    wid = s * NC + c
    r0 = s * RPT
    isx = (is0, is1, is2, is3, is4, is5)
    idx = (id0, id1, id2, id3, id4, id5)
    rows = (rows0, rows1)
    isem = (im0, im1, im2, im3, im4, im5)
    gsem = (gm0, gm1)
    ssem = (sm0, sm1)
    pltpu.sync_copy(zeros_hbm.at[pl.ds(r0, RPT)], acc.at[pl.ds(r0, RPT)])
    plsc.subcore_barrier()

    def e0_of(t):
        return (wid * TW + t) * K

    def load_idx(t, q):
        pltpu.async_copy(src_hbm.at[pl.ds(e0_of(t), K)], isx[q], isem[q])
        pltpu.async_copy(dst_hbm.at[pl.ds(e0_of(t), K)], idx[q], isem[q])

    def wait_idx(q):
        pltpu.make_async_copy(src_hbm.at[pl.ds(0, K)], isx[q], isem[q]).wait()
        pltpu.make_async_copy(dst_hbm.at[pl.ds(0, K)], idx[q], isem[q]).wait()

    def start_gather(r, q):
        pltpu.async_copy(hs_hbm.at[isx[q]], rows[r], gsem[r])

    def wait_gather(r, q):
        pltpu.make_async_copy(hs_hbm.at[isx[q]], rows[r], gsem[r]).wait()

    def start_scatter(r, q):
        pltpu.async_copy(rows[r], acc.at[idx[q]], ssem[r], add=True)

    def wait_scatter(r, q):
        pltpu.make_async_copy(rows[r], acc.at[idx[q]], ssem[r]).wait()

    for q in range(4):
        load_idx(q, q)
    wait_idx(0)
    start_gather(0, 0)

    def body(i, carry):
        for k in range(6):
            t = 6 * i + k
            q = k
            r = k % 2
            qn = (k + 1) % 6
            rn = (k + 1) % 2

            @pl.when(t >= 1)
            def _():
                wait_scatter(rn, (k + 5) % 6)

            @pl.when(t + 1 < TW)
            def _():
                wait_idx(qn)
                start_gather(rn, qn)

            wait_gather(r, q)
            start_scatter(r, q)

            @pl.when(t + 4 < TW)
            def _():
                load_idx(t + 4, (k + 4) % 6)

        return carry

    lax.fori_loop(0, TW // 6, body, 0)
    wait_scatter((TW - 1) % 2, (TW - 1) % 6)

    @pl.when(wid < EC - TW * NW)
    def _():
        e0 = (TW * NW + wid) * K
        pltpu.sync_copy(src_hbm.at[pl.ds(e0, K)], is0)
        pltpu.sync_copy(dst_hbm.at[pl.ds(e0, K)], id0)
        pltpu.async_copy(hs_hbm.at[is0], rows0, gm0).wait()
        pltpu.sync_copy(rows0, acc.at[id0], add=True)

    plsc.subcore_barrier()
    pltpu.sync_copy(acc.at[pl.ds(r0, RPT)], out_hbm.at[c, pl.ds(r0, RPT)])


@functools.partial(
    pl.kernel,
    mesh=_mesh,
    out_type=(jax.ShapeDtypeStruct((PP, D), jnp.float32),
              jax.ShapeDtypeStruct((PP, D), jnp.float32)),
    scratch_types=(
        [pltpu.VMEM((PCW * K,), jnp.int32)] * 2
        + [pltpu.VMEM((K, D), jnp.float32)] * 2
        + [pltpu.SemaphoreType.DMA] * 2
    ),
)
def _sc_pair_gather(ha_hbm, hb_hbm, u_hbm, v_hbm, outa_hbm, outb_hbm,
                    uidx, vidx, rows0, rows1, gm0, gm1):
    c = lax.axis_index("c")
    s = lax.axis_index("s")
    wid = s * NC + c
    p0 = wid * PCW * K
    rows = (rows0, rows1)
    gsem = (gm0, gm1)
    pltpu.sync_copy(u_hbm.at[pl.ds(p0, PCW * K)], uidx)
    pltpu.sync_copy(v_hbm.at[pl.ds(p0, PCW * K)], vidx)

    def src_of(unit):
        j = unit // 2
        if unit % 2 == 0:
            return ha_hbm, uidx.at[pl.ds(j * K, K)], outa_hbm
        return hb_hbm, vidx.at[pl.ds(j * K, K)], outb_hbm

    def start(unit, r):
        tab, ix, _ = src_of(unit)
        pltpu.async_copy(tab.at[ix], rows[r], gsem[r])

    def finish(unit, r):
        tab, ix, out = src_of(unit)
        pltpu.make_async_copy(tab.at[ix], rows[r], gsem[r]).wait()
        pltpu.sync_copy(rows[r], out.at[pl.ds(p0 + (unit // 2) * K, K)])

    start(0, 0)
    for unit in range(2 * PCW):
        if unit + 1 < 2 * PCW:
            start(unit + 1, (unit + 1) % 2)
        finish(unit, unit % 2)


def _rsqrt_deg(a_ref, b_ref, col):
    deg = a_ref[:, col:col + 1] + b_ref[:, col:col + 1]
    return lax.rsqrt(jnp.maximum(deg, 1.0))


def _enc_body(t_ref, emb_ref, aw_ref, ab_ref, d0_ref, d1_ref, o_ref):
    t = t_ref[...]
    oh = (t == lax.broadcasted_iota(jnp.int32, (BR, D), 1)).astype(jnp.float32)
    embw = jnp.dot(emb_ref[...], aw_ref[...], preferred_element_type=jnp.float32)
    h = jnp.dot(oh, embw, preferred_element_type=jnp.float32) + ab_ref[...]
    o_ref[...] = h * _rsqrt_deg(d0_ref, d1_ref, 0)


def _layer_body(p0_ref, p1_ref, d0_ref, d1_ref, w_ref, b_ref, o_ref):
    agg = (p0_ref[...] + p1_ref[...]) * _rsqrt_deg(d0_ref, d1_ref, 64)
    h = jnp.dot(agg, w_ref[...], preferred_element_type=jnp.float32) + b_ref[...]
    h = jnp.maximum(h, 0.0)
    o_ref[...] = h * _rsqrt_deg(d0_ref, d1_ref, 0)


def _layer3_body(p0_ref, p1_ref, d0_ref, d1_ref, w_ref, b_ref,
                 w1a_ref, w1b_ref, b1_ref, oa_ref, ob_ref):
    agg = (p0_ref[...] + p1_ref[...]) * _rsqrt_deg(d0_ref, d1_ref, 64)
    h = jnp.dot(agg, w_ref[...], preferred_element_type=jnp.float32) + b_ref[...]
    h = jnp.maximum(h, 0.0)
    oa_ref[...] = jnp.dot(h, w1a_ref[...], preferred_element_type=jnp.float32) + b1_ref[...]
    ob_ref[...] = jnp.dot(h, w1b_ref[...], preferred_element_type=jnp.float32)


def _score_body(a_ref, b_ref, w2_ref, b2_ref, o_ref):
    hid = jnp.maximum(a_ref[...] + b_ref[...], 0.0)
    s = jnp.sum(hid * w2_ref[...], axis=1, keepdims=True)
    o_ref[...] = s + b2_ref[:, 0:1]


def _row_spec(bs):
    return pl.BlockSpec(bs, lambda i: (i, 0))


def _full_spec(shape):
    return pl.BlockSpec(shape, lambda i: (0, 0))


def _tc_encoder(types2, emb_pad, atom_W, atom_b2, dg0, dg1):
    return pl.pallas_call(
        _enc_body,
        grid=(N // BR,),
        in_specs=[_row_spec((BR, 1)), _full_spec((D, D)), _full_spec((D, D)),
                  _full_spec((1, D)), _row_spec((BR, D)), _row_spec((BR, D))],
        out_specs=_row_spec((BR, D)),
        out_shape=jax.ShapeDtypeStruct((N, D), jnp.float32),
    )(types2, emb_pad, atom_W, atom_b2, dg0, dg1)


def _tc_layer(p0, p1, dg0, dg1, w, b2):
    return pl.pallas_call(
        _layer_body,
        grid=(N // BR,),
        in_specs=[_row_spec((BR, D)), _row_spec((BR, D)),
                  _row_spec((BR, D)), _row_spec((BR, D)),
                  _full_spec((D, D)), _full_spec((1, D))],
        out_specs=_row_spec((BR, D)),
        out_shape=jax.ShapeDtypeStruct((N, D), jnp.float32),
    )(p0, p1, dg0, dg1, w, b2)


def _tc_layer3(p0, p1, dg0, dg1, w, b2, w1a, w1b, b12):
    return pl.pallas_call(
        _layer3_body,
        grid=(N // BR,),
        in_specs=[_row_spec((BR, D)), _row_spec((BR, D)),
                  _row_spec((BR, D)), _row_spec((BR, D)),
                  _full_spec((D, D)), _full_spec((1, D)),
                  _full_spec((D, D)), _full_spec((D, D)), _full_spec((1, D))],
        out_specs=(_row_spec((BR, D)), _row_spec((BR, D))),
        out_shape=(jax.ShapeDtypeStruct((N, D), jnp.float32),
                   jax.ShapeDtypeStruct((N, D), jnp.float32)),
    )(p0, p1, dg0, dg1, w, b2, w1a, w1b, b12)


def _tc_score(hag, hbg, w2row, b2row):
    return pl.pallas_call(
        _score_body,
        grid=(PP // BP,),
        in_specs=[_row_spec((BP, D)), _row_spec((BP, D)),
                  _full_spec((1, D)), _full_spec((1, D))],
        out_specs=_row_spec((BP, 1)),
        out_shape=jax.ShapeDtypeStruct((PP, 1), jnp.float32),
    )(hag, hbg, w2row, b2row)


def kernel(atom_types, bond_types, edge_index, candidates,
           atom_embed, atom_W, atom_b,
           bond_embed, bond_W, bond_b,
           conv_W0, conv_b0, conv_W1, conv_b1, conv_W2, conv_b2,
           sc_W1, sc_b1, sc_W2, sc_b2):
    f32 = jnp.float32
    src = edge_index[0].astype(jnp.int32)
    dst = edge_index[1].astype(jnp.int32)
    types2 = atom_types.astype(jnp.int32).reshape(N, 1)
    emb_pad = jnp.pad(atom_embed, ((0, D - atom_embed.shape[0]), (0, 0)))

    zerosND = jnp.zeros((NP, D), f32)
    col = jnp.arange(D, dtype=jnp.int32)
    ones_s = jnp.broadcast_to((col == 0).astype(f32), (K, D))
    ones_d = jnp.broadcast_to((col == 64).astype(f32), (K, D))

    degp = _sc_degrees(src, dst, ones_s, ones_d, zerosND)
    dg0, dg1 = degp[0], degp[1]

    h = _tc_encoder(types2, emb_pad, atom_W, atom_b.reshape(1, D), dg0, dg1)

    agg = _sc_aggregate(h, src, dst, zerosND)
    h = _tc_layer(agg[0], agg[1], dg0, dg1, conv_W0, conv_b0.reshape(1, D))
    agg = _sc_aggregate(h, src, dst, zerosND)
    h = _tc_layer(agg[0], agg[1], dg0, dg1, conv_W1, conv_b1.reshape(1, D))
    agg = _sc_aggregate(h, src, dst, zerosND)

    ha, hb = _tc_layer3(agg[0], agg[1], dg0, dg1,
                        conv_W2, conv_b2.reshape(1, D),
                        sc_W1[:D], sc_W1[D:], sc_b1.reshape(1, D))

    pad = (jnp.arange(PP - P, dtype=jnp.int32) * 37) % N
    u = jnp.concatenate([candidates[:, 0].astype(jnp.int32), pad])
    v = jnp.concatenate([candidates[:, 1].astype(jnp.int32), pad])

    hag, hbg = _sc_pair_gather(ha, hb, u, v)

    w2row = sc_W2.reshape(1, D)
    b2row = jnp.broadcast_to(sc_b2.reshape(1, 1), (1, D))
    scores = _tc_score(hag, hbg, w2row, b2row)
    return scores[:P, 0]
